# baseline (speedup 1.0000x reference)
import functools
import jax
import jax.numpy as jnp
from jax import lax
from jax.experimental import pallas as pl
from jax.experimental.pallas import tpu as pltpu, tpu_sc as plsc

_D = 64
_N = 16384
_S = 50


def _make_gather(N, S, D):
  info = plsc.get_sparse_core_info()
  NC, NS = info.num_cores, info.num_subcores
  NW = NC * NS
  assert N % NW == 0
  n_per_w = N // NW
  R = 8
  NBUF = 4
  assert n_per_w % (R * NBUF) == 0
  n_chunks = n_per_w // R

  mesh = plsc.VectorSubcoreMesh(core_axis_name="c", subcore_axis_name="s")

  @functools.partial(
      pl.kernel,
      out_type=jax.ShapeDtypeStruct((N, S, D), jnp.float32),
      mesh=mesh,
      compiler_params=pltpu.CompilerParams(use_tc_tiling_on_sc=False),
      scratch_types=[
          pltpu.VMEM((n_per_w, S), jnp.int32),
          pltpu.VMEM((NBUF, R, S, D), jnp.float32),
          [pltpu.SemaphoreType.DMA] * NBUF,
          [pltpu.SemaphoreType.DMA] * NBUF,
      ],
  )
  def gather_kernel(x_hbm, table_hbm, out_hbm, idx_v, rows_v, gsems, wsems):
    wid = lax.axis_index("s") * NC + lax.axis_index("c")
    base = wid * n_per_w
    pltpu.sync_copy(x_hbm.at[pl.ds(base, n_per_w)], idx_v)

    def start_gather(c, b):
      for i in range(R):
        pltpu.async_copy(
            table_hbm.at[idx_v.at[c * R + i]], rows_v.at[b, i], gsems[b])

    def wait_gather(b):
      for i in range(R):
        pltpu.make_async_copy(
            table_hbm.at[idx_v.at[0]], rows_v.at[b, i], gsems[b]).wait()

    def start_write(c, b):
      pltpu.async_copy(
          rows_v.at[b], out_hbm.at[pl.ds(base + c * R, R)], wsems[b])

    def wait_write(b):
      pltpu.make_async_copy(
          rows_v.at[b], out_hbm.at[pl.ds(base, R)], wsems[b]).wait()

    for c in range(NBUF - 1):
      start_gather(c, c)

    for j in range(NBUF):
      wait_gather(j)
      start_write(j, j)
      pf = j + NBUF - 1
      if pf >= NBUF:
        wait_write(pf % NBUF)
      start_gather(pf, pf % NBUF)

    @pl.loop(1, n_chunks // NBUF - 1)
    def _group(i):
      c0 = i * NBUF
      for j in range(NBUF):
        c = c0 + j
        wait_gather(j)
        start_write(c, j)
        wait_write((j + NBUF - 1) % NBUF)
        start_gather(c + NBUF - 1, (j + NBUF - 1) % NBUF)

    c0 = n_chunks - NBUF
    for j in range(NBUF):
      c = c0 + j
      wait_gather(j)
      start_write(c, j)
      if c + NBUF - 1 < n_chunks:
        wait_write((j + NBUF - 1) % NBUF)
        start_gather(c + NBUF - 1, (j + NBUF - 1) % NBUF)
    for j in range(NBUF):
      wait_write(j)

  return gather_kernel


_gather = _make_gather(_N, _S, _D)


def kernel(x, table):
  return _gather(x, table)


# speedup vs baseline: 1.2339x; 1.2339x over previous
import functools
import jax
import jax.numpy as jnp
from jax import lax
from jax.experimental import pallas as pl
from jax.experimental.pallas import tpu as pltpu, tpu_sc as plsc

_D = 64
_DP = 128
_N = 16384
_S = 50
_SP = 56


def _make_gather(N, S, D):
  info = plsc.get_sparse_core_info()
  NC, NS = info.num_cores, info.num_subcores
  NW = NC * NS
  assert N % NW == 0
  n_per_w = N // NW
  R = 8
  NBUF = 2
  assert n_per_w % (R * NBUF) == 0
  n_chunks = n_per_w // R

  mesh = plsc.VectorSubcoreMesh(core_axis_name="c", subcore_axis_name="s")

  @functools.partial(
      pl.kernel,
      out_type=jax.ShapeDtypeStruct((N, _SP, _DP), jnp.float32),
      mesh=mesh,
      compiler_params=pltpu.CompilerParams(use_tc_tiling_on_sc=False),
      scratch_types=[
          pltpu.VMEM((n_per_w, S), jnp.int32),
          pltpu.VMEM((NBUF, R, S, _DP), jnp.float32),
          [pltpu.SemaphoreType.DMA] * NBUF,
          [pltpu.SemaphoreType.DMA] * NBUF,
      ],
  )
  def gather_kernel(x_hbm, table_hbm, out_hbm, idx_v, rows_v, gsems, wsems):
    wid = lax.axis_index("s") * NC + lax.axis_index("c")
    base = wid * n_per_w
    pltpu.sync_copy(x_hbm.at[pl.ds(base, n_per_w)], idx_v)

    def start_gather(c, b):
      for i in range(R):
        pltpu.async_copy(
            table_hbm.at[idx_v.at[c * R + i]], rows_v.at[b, i], gsems[b])

    def wait_gather(b):
      for i in range(R):
        pltpu.make_async_copy(
            table_hbm.at[idx_v.at[0]], rows_v.at[b, i], gsems[b]).wait()

    def start_write(c, b):
      pltpu.async_copy(
          rows_v.at[b],
          out_hbm.at[pl.ds(base + c * R, R), pl.ds(0, S)], wsems[b])

    def wait_write(b):
      pltpu.make_async_copy(
          rows_v.at[b],
          out_hbm.at[pl.ds(base, R), pl.ds(0, S)], wsems[b]).wait()

    for c in range(NBUF - 1):
      start_gather(c, c)

    for j in range(NBUF):
      wait_gather(j)
      start_write(j, j)
      pf = j + NBUF - 1
      if pf >= NBUF:
        wait_write(pf % NBUF)
      start_gather(pf, pf % NBUF)

    @pl.loop(1, n_chunks // NBUF - 1)
    def _group(i):
      c0 = i * NBUF
      for j in range(NBUF):
        c = c0 + j
        wait_gather(j)
        start_write(c, j)
        wait_write((j + NBUF - 1) % NBUF)
        start_gather(c + NBUF - 1, (j + NBUF - 1) % NBUF)

    c0 = n_chunks - NBUF
    for j in range(NBUF):
      c = c0 + j
      wait_gather(j)
      start_write(c, j)
      if c + NBUF - 1 < n_chunks:
        wait_write((j + NBUF - 1) % NBUF)
        start_gather(c + NBUF - 1, (j + NBUF - 1) % NBUF)
    for j in range(NBUF):
      wait_write(j)

  return gather_kernel


_gather = _make_gather(_N, _S, _D)


def kernel(x, table):
  table_p = jnp.pad(table, ((0, 0), (0, _DP - _D)))
  out_p = _gather(x, table_p)
  return out_p[:, :_S, :_D]


# speedup vs baseline: 1.4414x; 1.1682x over previous
import functools
import jax
import jax.numpy as jnp
from jax import lax
from jax.experimental import pallas as pl
from jax.experimental.pallas import tpu as pltpu, tpu_sc as plsc

_D = 64
_DP = 128
_N = 16384
_S = 50
_SP = 56


def _make_gather(N, S, D):
  info = plsc.get_sparse_core_info()
  NC, NS = info.num_cores, info.num_subcores
  NW = NC * NS
  assert N % NW == 0
  n_per_w = N // NW
  R = 8
  NBUF = 2
  assert n_per_w % (R * NBUF) == 0
  n_chunks = n_per_w // R

  mesh = plsc.VectorSubcoreMesh(core_axis_name="c", subcore_axis_name="s")

  @functools.partial(
      pl.kernel,
      out_type=jax.ShapeDtypeStruct((N, _SP, _DP), jnp.float32),
      mesh=mesh,
      compiler_params=pltpu.CompilerParams(use_tc_tiling_on_sc=False),
      scratch_types=[
          pltpu.VMEM((n_per_w, S), jnp.int32),
          pltpu.VMEM((NBUF, R, S, _D), jnp.float32),
          [pltpu.SemaphoreType.DMA] * NBUF,
          [pltpu.SemaphoreType.DMA] * NBUF,
      ],
  )
  def gather_kernel(x_hbm, table_hbm, out_hbm, idx_v, rows_v, gsems, wsems):
    wid = lax.axis_index("s") * NC + lax.axis_index("c")
    base = wid * n_per_w
    pltpu.sync_copy(x_hbm.at[pl.ds(base, n_per_w)], idx_v)

    def start_gather(c, b):
      for i in range(R):
        pltpu.async_copy(
            table_hbm.at[idx_v.at[c * R + i]], rows_v.at[b, i], gsems[b])

    def wait_gather(b):
      for i in range(R):
        pltpu.make_async_copy(
            table_hbm.at[idx_v.at[0]], rows_v.at[b, i], gsems[b]).wait()

    def start_write(c, b):
      pltpu.async_copy(
          rows_v.at[b],
          out_hbm.at[pl.ds(base + c * R, R), pl.ds(0, S), pl.ds(0, _D)],
          wsems[b])

    def wait_write(b):
      pltpu.make_async_copy(
          rows_v.at[b],
          out_hbm.at[pl.ds(base, R), pl.ds(0, S), pl.ds(0, _D)],
          wsems[b]).wait()

    for c in range(NBUF - 1):
      start_gather(c, c)

    for j in range(NBUF):
      wait_gather(j)
      start_write(j, j)
      pf = j + NBUF - 1
      if pf >= NBUF:
        wait_write(pf % NBUF)
      start_gather(pf, pf % NBUF)

    @pl.loop(1, n_chunks // NBUF - 1)
    def _group(i):
      c0 = i * NBUF
      for j in range(NBUF):
        c = c0 + j
        wait_gather(j)
        start_write(c, j)
        wait_write((j + NBUF - 1) % NBUF)
        start_gather(c + NBUF - 1, (j + NBUF - 1) % NBUF)

    c0 = n_chunks - NBUF
    for j in range(NBUF):
      c = c0 + j
      wait_gather(j)
      start_write(c, j)
      if c + NBUF - 1 < n_chunks:
        wait_write((j + NBUF - 1) % NBUF)
        start_gather(c + NBUF - 1, (j + NBUF - 1) % NBUF)
    for j in range(NBUF):
      wait_write(j)

  return gather_kernel


_gather = _make_gather(_N, _S, _D)


def kernel(x, table):
  table_p = jnp.pad(table, ((0, 0), (0, _DP - _D)))
  table_v = table_p.reshape(2 * table.shape[0], _D)
  out_p = _gather(x * 2, table_v)
  return out_p[:, :_S, :_D]


# speedup vs baseline: 1.4477x; 1.0044x over previous
import functools
import jax
import jax.numpy as jnp
from jax import lax
from jax.experimental import pallas as pl
from jax.experimental.pallas import tpu as pltpu, tpu_sc as plsc

_D = 64
_DP = 128
_N = 16384
_S = 50
_SP = 56


def _make_gather(N, S, D):
  info = plsc.get_sparse_core_info()
  NC, NS = info.num_cores, info.num_subcores
  NW = NC * NS
  assert N % NW == 0
  n_per_w = N // NW
  R = 8
  NBUF = 4
  assert n_per_w % (R * NBUF) == 0
  n_chunks = n_per_w // R

  mesh = plsc.VectorSubcoreMesh(core_axis_name="c", subcore_axis_name="s")

  @functools.partial(
      pl.kernel,
      out_type=jax.ShapeDtypeStruct((N, _SP, _DP), jnp.float32),
      mesh=mesh,
      compiler_params=pltpu.CompilerParams(use_tc_tiling_on_sc=False),
      scratch_types=[
          pltpu.VMEM((n_per_w, S), jnp.int32),
          pltpu.VMEM((NBUF, R, S, _D), jnp.float32),
          [pltpu.SemaphoreType.DMA] * NBUF,
          [pltpu.SemaphoreType.DMA] * NBUF,
      ],
  )
  def gather_kernel(x_hbm, table_hbm, out_hbm, idx_v, rows_v, gsems, wsems):
    wid = lax.axis_index("s") * NC + lax.axis_index("c")
    base = wid * n_per_w
    pltpu.sync_copy(x_hbm.at[pl.ds(base, n_per_w)], idx_v)

    def start_gather(c, b):
      for i in range(R):
        pltpu.async_copy(
            table_hbm.at[idx_v.at[c * R + i]], rows_v.at[b, i], gsems[b])

    def wait_gather(b):
      for i in range(R):
        pltpu.make_async_copy(
            table_hbm.at[idx_v.at[0]], rows_v.at[b, i], gsems[b]).wait()

    def start_write(c, b):
      pltpu.async_copy(
          rows_v.at[b],
          out_hbm.at[pl.ds(base + c * R, R), pl.ds(0, S), pl.ds(0, _D)],
          wsems[b])

    def wait_write(b):
      pltpu.make_async_copy(
          rows_v.at[b],
          out_hbm.at[pl.ds(base, R), pl.ds(0, S), pl.ds(0, _D)],
          wsems[b]).wait()

    for c in range(NBUF - 1):
      start_gather(c, c)

    for j in range(NBUF):
      wait_gather(j)
      start_write(j, j)
      pf = j + NBUF - 1
      if pf >= NBUF:
        wait_write(pf % NBUF)
      start_gather(pf, pf % NBUF)

    @pl.loop(1, n_chunks // NBUF - 1)
    def _group(i):
      c0 = i * NBUF
      for j in range(NBUF):
        c = c0 + j
        wait_gather(j)
        start_write(c, j)
        wait_write((j + NBUF - 1) % NBUF)
        start_gather(c + NBUF - 1, (j + NBUF - 1) % NBUF)

    c0 = n_chunks - NBUF
    for j in range(NBUF):
      c = c0 + j
      wait_gather(j)
      start_write(c, j)
      if c + NBUF - 1 < n_chunks:
        wait_write((j + NBUF - 1) % NBUF)
        start_gather(c + NBUF - 1, (j + NBUF - 1) % NBUF)
    for j in range(NBUF):
      wait_write(j)

  return gather_kernel


_gather = _make_gather(_N, _S, _D)


def kernel(x, table):
  table_p = jnp.pad(table, ((0, 0), (0, _DP - _D)))
  table_v = table_p.reshape(2 * table.shape[0], _D)
  out_p = _gather(x * 2, table_v)
  return out_p[:, :_S, :_D]
